# Initial kernel scaffold; baseline (speedup 1.0000x reference)
import jax
import jax.numpy as jnp
from jax.experimental import pallas as pl


def kernel(x, table):
    raise NotImplementedError("write your pallas kernel here")


# speedup vs baseline: 2.0813x; 2.0813x over previous
import functools

import jax
import jax.numpy as jnp
from jax import lax
from jax.experimental import pallas as pl
from jax.experimental.pallas import tpu as pltpu
from jax.experimental.pallas import tpu_sc as plsc

EMB = 32
B = 4096
L = 200

NC = 2
NS = 16
NW = NC * NS
SPW = B // NW
TOK = SPW * L
NFULL = L // 16
TAIL = L - NFULL * 16


def _sc_kernel(x_hbm, table_hbm, out_hbm, idx_v, sid_v, rows_v, out_v,
               sem_g1, sem_g2):
    c = lax.axis_index("c")
    s = lax.axis_index("s")
    wid = s * NC + c
    base_tok = wid * TOK

    pltpu.sync_copy(x_hbm.at[pl.ds(base_tok, TOK)], idx_v.at[pl.ds(0, TOK)])

    lane = lax.iota(jnp.int32, 16)

    def sentence(si, carry):
        sbase = si * L
        cnt = jnp.zeros((16,), jnp.int32)
        for k in range(NFULL + 1):
            v = idx_v[pl.ds(sbase + 16 * k, 16)]
            xm = jnp.where(v == 1, 0, v)
            if k == NFULL:
                valid = (xm != 0) & (lane < TAIL)
            else:
                valid = xm != 0
            cnt = cnt + plsc.all_reduce_population_count(valid)
            sid_v[pl.ds(16 * k, 16)] = xm
        count = cnt.astype(jnp.float32)

        cp1 = pltpu.async_copy(table_hbm.at[sid_v.at[pl.ds(0, 128)]],
                               rows_v.at[pl.ds(0, 128)], sem_g1)
        cp2 = pltpu.async_copy(table_hbm.at[sid_v.at[pl.ds(128, 72)]],
                               rows_v.at[pl.ds(128, 72)], sem_g2)
        cp1.wait()
        cp2.wait()

        def red(o, accs):
            a0, a1 = accs
            for j in range(8):
                r = o * 8 + j
                a0 = a0 + rows_v[r, pl.ds(0, 16)]
                a1 = a1 + rows_v[r, pl.ds(16, 16)]
            return a0, a1

        acc0, acc1 = lax.fori_loop(
            0, L // 8, red,
            (jnp.zeros((16,), jnp.float32), jnp.zeros((16,), jnp.float32)))

        scale = jnp.where(count > 0.0, 1.0 / jnp.maximum(count, 1.0), 0.0)
        out_v[si, pl.ds(0, 16)] = acc0 * scale
        out_v[si, pl.ds(16, 16)] = acc1 * scale
        return carry

    lax.fori_loop(0, SPW, sentence, 0)

    pltpu.sync_copy(out_v, out_hbm.at[pl.ds(wid * SPW, SPW)])


@jax.jit
def _run(x_flat, table):
    mesh = plsc.VectorSubcoreMesh(core_axis_name="c", subcore_axis_name="s")
    kern = functools.partial(
        pl.kernel,
        out_type=jax.ShapeDtypeStruct((B, EMB), jnp.float32),
        mesh=mesh,
        compiler_params=pltpu.CompilerParams(needs_layout_passes=False,
                                             use_tc_tiling_on_sc=False),
        scratch_types=[
            pltpu.VMEM((TOK + 16,), jnp.int32),
            pltpu.VMEM((208,), jnp.int32),
            pltpu.VMEM((L, EMB), jnp.float32),
            pltpu.VMEM((SPW, EMB), jnp.float32),
            pltpu.SemaphoreType.DMA,
            pltpu.SemaphoreType.DMA,
        ],
    )(_sc_kernel)
    return kern(x_flat, table)


def kernel(x, table):
    return _run(x.reshape(-1), table)
